# baseline (speedup 1.0000x reference)
import functools

import jax
import jax.numpy as jnp
from jax import lax
from jax.experimental import pallas as pl
from jax.experimental.pallas import tpu as pltpu
from jax.experimental.pallas import tpu_sc as plsc

B = 8
T_IN = 4096
D = 512
CTX = 5
T_OUT = 1365
TR = 5
NCHUNK = T_OUT // TR
SR = 24
NW = 32
MAXC = -(-NCHUNK // NW)
LANES = 16

_mesh = plsc.VectorSubcoreMesh(core_axis_name="c", subcore_axis_name="s")


@functools.partial(
    pl.kernel,
    mesh=_mesh,
    out_type=jax.ShapeDtypeStruct((T_OUT, B, CTX * D), jnp.float32),
    scratch_types=[
        pltpu.VMEM((SR, D), jnp.float32),
        pltpu.VMEM((SR, D), jnp.float32),
        pltpu.VMEM((TR, B, CTX * D), jnp.float32),
        pltpu.SemaphoreType.DMA,
        pltpu.SemaphoreType.DMA,
        pltpu.SemaphoreType.DMA,
    ],
)
def _splice(feats_hbm, out_hbm, slab0, slab1, obuf, lsem0, lsem1, wsem):
    nc = 2
    wid = lax.axis_index("s") * nc + lax.axis_index("c")
    slabs = (slab0, slab1)
    lsems = (lsem0, lsem1)

    def chunk_coords(k):
        c = wid + k * NW
        t0 = c * TR
        r0a = pl.multiple_of(
            jnp.clip((3 * t0 - 2) // 8 * 8, 0, T_IN - SR), 8)
        return c, t0, r0a

    def start_load(k, b, slab, lsem):
        c, _, r0a = chunk_coords(k)

        @pl.when(c < NCHUNK)
        def _():
            pltpu.make_async_copy(feats_hbm.at[b, pl.ds(r0a, SR)],
                                  slab, lsem).start()

    def process_chunk(k):
        c, t0, r0a = chunk_coords(k)

        @pl.when((k >= 1) & (c - NW < NCHUNK))
        def _():
            pltpu.make_async_copy(out_hbm.at[pl.ds(0, TR)], obuf,
                                  wsem).wait()

        @pl.when(c < NCHUNK)
        def _():
            for b in range(B):
                pltpu.make_async_copy(feats_hbm.at[b, pl.ds(r0a, SR)],
                                      slabs[b % 2], lsems[b % 2]).wait()
                if b + 1 < B:
                    pltpu.make_async_copy(
                        feats_hbm.at[b + 1, pl.ds(r0a, SR)],
                        slabs[(b + 1) % 2], lsems[(b + 1) % 2]).start()
                slab = slabs[b % 2]

                def row_body(tp, carry):
                    t = t0 + tp
                    for rr in range(CTX):
                        row = jnp.maximum(3 * t + rr - 2, 0) - r0a
                        for cc in range(D // LANES):
                            obuf[tp, b,
                                 pl.ds(rr * D + cc * LANES, LANES)] = (
                                slab[row, pl.ds(cc * LANES, LANES)])
                    return carry

                lax.fori_loop(0, TR, row_body, 0)

            pltpu.make_async_copy(obuf, out_hbm.at[pl.ds(t0, TR)],
                                  wsem).start()

        start_load(k + 1, 0, slabs[0], lsems[0])

    def body(i, carry):
        process_chunk(i)
        return carry

    start_load(0, 0, slabs[0], lsems[0])
    lax.fori_loop(0, MAXC, body, 0)

    c_last, _, _ = chunk_coords(MAXC - 1)

    @pl.when(c_last < NCHUNK)
    def _():
        pltpu.make_async_copy(out_hbm.at[pl.ds(0, TR)], obuf, wsem).wait()


def kernel(feats):
    o = _splice(feats)
    return jnp.transpose(o, (1, 0, 2))
